# baseline (device time: 19533 ns/iter reference)
import jax
import jax.numpy as jnp
from jax import lax
from jax.experimental import pallas as pl
from jax.experimental.pallas import tpu as pltpu

N_DEV = 4
B, SQ, SKV, HQ_LOCAL, DH = 2, 128, 128, 4, 64
D_MODEL = 512
ROWS = B * SQ
SCALE = 0.125


def _body(x_ref, wq_ref, k_ref, v_ref, wo_ref, out_ref,
          send_ref, recv1_ref, recv2_ref, send_sems, recv_sems):
    my_pos = lax.axis_index("i")
    p1 = my_pos ^ 1
    p2 = my_pos ^ 2

    barrier_sem = pltpu.get_barrier_semaphore()
    for nbr in (p1, p2):
        pl.semaphore_signal(
            barrier_sem, inc=1,
            device_id=(nbr,), device_id_type=pl.DeviceIdType.MESH,
        )
    pl.semaphore_wait(barrier_sem, 2)

    q = (jnp.dot(x_ref[...], wq_ref[...], preferred_element_type=jnp.float32)
         * SCALE).astype(jnp.bfloat16)

    ctx_rows = []
    for b in range(B):
        ctx_cols = []
        for h in range(HQ_LOCAL):
            qh = q[b * SQ:(b + 1) * SQ, h * DH:(h + 1) * DH]
            s = lax.dot_general(qh, k_ref[b, h], (((1,), (1,)), ((), ())),
                                preferred_element_type=jnp.float32)
            m = jnp.max(s, axis=-1, keepdims=True)
            w = jnp.exp(s - m)
            w = w / jnp.sum(w, axis=-1, keepdims=True)
            ctx = jnp.dot(w.astype(jnp.bfloat16), v_ref[b, h],
                          preferred_element_type=jnp.float32)
            ctx_cols.append(ctx.astype(jnp.bfloat16))
        ctx_rows.append(jnp.concatenate(ctx_cols, axis=1))
    ctx_all = jnp.concatenate(ctx_rows, axis=0)

    partial = jnp.dot(ctx_all, wo_ref[...], preferred_element_type=jnp.float32)
    out_ref[...] = partial
    send_ref[...] = partial.astype(jnp.bfloat16)

    r1 = pltpu.make_async_remote_copy(
        src_ref=send_ref, dst_ref=recv1_ref,
        send_sem=send_sems.at[0], recv_sem=recv_sems.at[0],
        device_id=(p1,), device_id_type=pl.DeviceIdType.MESH,
    )
    r1.start()
    r1.wait()
    s1 = out_ref[...] + recv1_ref[...].astype(jnp.float32)
    out_ref[...] = s1
    send_ref[...] = s1.astype(jnp.bfloat16)

    r2 = pltpu.make_async_remote_copy(
        src_ref=send_ref, dst_ref=recv2_ref,
        send_sem=send_sems.at[1], recv_sem=recv_sems.at[1],
        device_id=(p2,), device_id_type=pl.DeviceIdType.MESH,
    )
    r2.start()
    r2.wait()
    out_ref[...] = out_ref[...] + recv2_ref[...].astype(jnp.float32)


def kernel(x, Wq, K_ext, V_ext, Wo):
    my_pos = lax.axis_index("i")
    K_l = lax.dynamic_slice_in_dim(K_ext, my_pos * HQ_LOCAL, HQ_LOCAL, axis=2)
    V_l = lax.dynamic_slice_in_dim(V_ext, my_pos * HQ_LOCAL, HQ_LOCAL, axis=2)
    K_l = jnp.transpose(K_l, (0, 2, 1, 3)).astype(jnp.bfloat16)
    V_l = jnp.transpose(V_l, (0, 2, 1, 3)).astype(jnp.bfloat16)
    x2 = x.reshape(ROWS, D_MODEL).astype(jnp.bfloat16)
    comm_shape = (ROWS, D_MODEL)
    out = pl.pallas_call(
        _body,
        out_shape=jax.ShapeDtypeStruct(comm_shape, jnp.float32),
        in_specs=[pl.BlockSpec(memory_space=pltpu.VMEM)] * 5,
        out_specs=pl.BlockSpec(memory_space=pltpu.VMEM),
        scratch_shapes=[
            pltpu.VMEM(comm_shape, jnp.bfloat16),
            pltpu.VMEM(comm_shape, jnp.bfloat16),
            pltpu.VMEM(comm_shape, jnp.bfloat16),
            pltpu.SemaphoreType.DMA((2,)),
            pltpu.SemaphoreType.DMA((2,)),
        ],
        compiler_params=pltpu.CompilerParams(collective_id=0),
    )(x2, Wq.astype(jnp.bfloat16), K_l, V_l, Wo.astype(jnp.bfloat16))
    return out.reshape(B, SQ, D_MODEL)
